# baseline (speedup 1.0000x reference)
import functools

import jax
import jax.numpy as jnp
from jax import lax
from jax.experimental import pallas as pl
from jax.experimental.pallas import tpu as pltpu
from jax.experimental.pallas import tpu_sc as plsc

B = 16384
GLOBAL_VOCAB = 1000000
LOCAL_VOCAB = 100000
DIM = 32


@functools.lru_cache(maxsize=1)
def _build():
    info = plsc.get_sparse_core_info()
    NC, NS = info.num_cores, info.num_subcores
    NW = NC * NS
    bpw = B // NW
    mesh = plsc.VectorSubcoreMesh(core_axis_name="c", subcore_axis_name="s")

    @functools.partial(
        pl.kernel,
        mesh=mesh,
        compiler_params=pltpu.CompilerParams(use_tc_tiling_on_sc=False,
                                             needs_layout_passes=False),
        out_type=(
            jax.ShapeDtypeStruct((4 * B, 16), jnp.float32),
            jax.ShapeDtypeStruct((4 * B, 16), jnp.float32),
        ),
        scratch_types=[
            pltpu.VMEM((bpw,), jnp.int32),
            pltpu.VMEM((bpw,), jnp.int32),
            pltpu.VMEM((bpw,), jnp.int32),
            pltpu.VMEM((bpw,), jnp.int32),
            pltpu.VMEM((2 * bpw,), jnp.int32),
            pltpu.VMEM((2 * bpw,), jnp.int32),
            pltpu.VMEM((2 * bpw,), jnp.int32),
            pltpu.VMEM((2 * bpw,), jnp.int32),
            pltpu.VMEM((2 * bpw,), jnp.int32),
            pltpu.VMEM((2 * bpw,), jnp.int32),
            pltpu.VMEM((2 * bpw, 16), jnp.float32),
            pltpu.VMEM((2 * bpw, 16), jnp.float32),
            pltpu.VMEM((2 * bpw, 16), jnp.float32),
            pltpu.VMEM((2 * bpw, 16), jnp.float32),
            pltpu.SemaphoreType.DMA,
            pltpu.SemaphoreType.DMA,
        ],
    )
    def k(Wu, Wi, Wa, Wb, uid, iid, ca, cb, ou_hbm, oi_hbm, g_out, l_out,
          idx_u, idx_i, idx_a, idx_b,
          x2_u, x2_i, x2_a, x2_b, oidx_u, oidx_i,
          r_u, r_i, r_a, r_b, sem, sem2):
        wid = lax.axis_index("s") * NC + lax.axis_index("c")
        base = wid * bpw
        pltpu.sync_copy(uid.at[pl.ds(base, bpw)], idx_u)
        pltpu.sync_copy(iid.at[pl.ds(base, bpw)], idx_i)
        pltpu.sync_copy(ca.at[pl.ds(base, bpw)], idx_a)
        pltpu.sync_copy(cb.at[pl.ds(base, bpw)], idx_b)
        pltpu.sync_copy(ou_hbm.at[pl.ds(2 * base, 2 * bpw)], oidx_u)
        pltpu.sync_copy(oi_hbm.at[pl.ds(2 * base, 2 * bpw)], oidx_i)

        lane = lax.iota(jnp.int32, 16)

        def expand(c, _):
            pos = 32 * c + 2 * lane
            for src, dst in ((idx_u, x2_u), (idx_i, x2_i),
                             (idx_a, x2_a), (idx_b, x2_b)):
                v = 2 * src[pl.ds(c * 16, 16)]
                plsc.store_scatter(dst, [pos], v)
                plsc.store_scatter(dst, [pos + 1], v + 1)
            return _

        lax.fori_loop(0, bpw // 16, expand, 0)

        du = pltpu.async_copy(Wu.at[x2_u], r_u, sem)
        di = pltpu.async_copy(Wi.at[x2_i], r_i, sem)
        da = pltpu.async_copy(Wa.at[x2_a], r_a, sem)
        db = pltpu.async_copy(Wb.at[x2_b], r_b, sem)
        du.wait()
        su = pltpu.async_copy(r_u, g_out.at[oidx_u], sem2)
        di.wait()
        si = pltpu.async_copy(r_i, g_out.at[oidx_i], sem2)
        da.wait()
        sa = pltpu.async_copy(r_a, l_out.at[oidx_u], sem2)
        db.wait()
        sb = pltpu.async_copy(r_b, l_out.at[oidx_i], sem2)
        su.wait()
        si.wait()
        sa.wait()
        sb.wait()

    return k


def kernel(W_user, W_item, W_cat_a, W_cat_b, user_id, item_id, cat_a, cat_b):
    k = _build()
    b4 = 4 * jnp.arange(B, dtype=jnp.int32)
    ou = jnp.stack([b4, b4 + 1], axis=1).reshape(-1)
    oi = ou + 2
    g4, l4 = k(W_user.reshape(-1, 16), W_item.reshape(-1, 16),
               W_cat_a.reshape(-1, 16), W_cat_b.reshape(-1, 16),
               user_id.astype(jnp.int32), item_id.astype(jnp.int32),
               cat_a.astype(jnp.int32), cat_b.astype(jnp.int32),
               ou, oi)
    return (g4.reshape(B, 2 * DIM), l4.reshape(B, 2 * DIM))
